# Initial kernel scaffold; baseline (speedup 1.0000x reference)
import jax
import jax.numpy as jnp
from jax.experimental import pallas as pl


def kernel(attns):
    raise NotImplementedError("write your pallas kernel here")


# speedup vs baseline: 27.4528x; 27.4528x over previous
import functools

import numpy as np
import jax
import jax.numpy as jnp
from jax import lax
from jax.experimental import pallas as pl
from jax.experimental.pallas import tpu as pltpu
from jax.experimental.pallas import tpu_sc as plsc

B = 4
S = 2048
H = S // 2
NC = 2
NS = 16
NW = NC * NS
ROWS_PER = S // NW
PAD = 16

_k = np.arange(S, dtype=np.float64)[:, None]
_m = np.arange(1, H + 1, dtype=np.float64)[None, :]
_ang = (2.0 * np.pi / S) * _k * _m
_DFT_COS = np.cos(_ang).astype(np.float32)
_DFT_SIN = np.sin(_ang).astype(np.float32)
_INV_COUNTS = np.tile((1.0 / (S - np.arange(S))).astype(np.float32), B)[None, :]


def _sc_body(a_ref, out_ref, buf, acc):
    wid = 2 * lax.axis_index("s") + lax.axis_index("c")
    zeros16 = jnp.zeros((16,), jnp.float32)
    buf[pl.ds(S, 16)] = zeros16

    for b in range(B):
        def zero_body(t, carry):
            acc[pl.ds(t * 16, 16)] = zeros16
            return carry
        lax.fori_loop(0, (S + PAD) // 16, zero_body, 0)

        def row_body(r, carry):
            i = wid + NW * r
            pltpu.sync_copy(a_ref.at[b * S + i], buf.at[pl.ds(0, S)])
            nchunks = (S - i + 15) // 16

            def chunk_body(kk, c2):
                p = kk * 16
                acc[pl.ds(p, 16)] = acc[pl.ds(p, 16)] + buf[pl.ds(i + p, 16)]
                return c2

            lax.fori_loop(0, nchunks, chunk_body, 0)
            return carry

        lax.fori_loop(0, ROWS_PER, row_body, 0)
        pltpu.sync_copy(acc.at[pl.ds(0, S)], out_ref.at[wid, b])


@functools.partial(
    pl.kernel,
    out_type=jax.ShapeDtypeStruct((NW, B, S), jnp.float32),
    mesh=plsc.VectorSubcoreMesh(
        core_axis_name="c", subcore_axis_name="s", num_cores=NC, num_subcores=NS
    ),
    scratch_types=[
        pltpu.VMEM((S + PAD,), jnp.float32),
        pltpu.VMEM((S + PAD,), jnp.float32),
    ],
)
def _sc_diag(a_ref, out_ref, buf, acc):
    _sc_body(a_ref, out_ref, buf, acc)


def _reduce_body(p_ref, invc_ref, o_ref):
    o_ref[...] = jnp.sum(p_ref[...], axis=0, keepdims=True) * invc_ref[...]


def _finish_body(w_ref, c_ref, s_ref, o_ref):
    waves = w_ref[...]
    re = jnp.dot(waves, c_ref[...], precision=lax.Precision.HIGHEST)
    im = jnp.dot(waves, s_ref[...], precision=lax.Precision.HIGHEST)
    mag = jnp.sqrt(re * re + im * im)
    mx = jnp.max(mag, axis=1, keepdims=True)
    sm = jnp.sum(mag, axis=1, keepdims=True)
    jv = (1.0 - mx / sm) * (1.0 / B)
    o_ref[...] = jnp.sum(jv, axis=0, keepdims=True)


def kernel(attns):
    a2 = attns.reshape(B * S, S)
    partials = _sc_diag(a2)
    waves_flat = pl.pallas_call(
        _reduce_body,
        out_shape=jax.ShapeDtypeStruct((1, B * S), jnp.float32),
    )(partials.reshape(NW, B * S), jnp.asarray(_INV_COUNTS))
    waves = waves_flat.reshape(B, S)
    out = pl.pallas_call(
        _finish_body,
        out_shape=jax.ShapeDtypeStruct((1, 1), jnp.float32),
    )(waves, jnp.asarray(_DFT_COS), jnp.asarray(_DFT_SIN))
    return out[0, 0]


# speedup vs baseline: 60.6745x; 2.2101x over previous
import functools

import numpy as np
import jax
import jax.numpy as jnp
from jax import lax
from jax.experimental import pallas as pl
from jax.experimental.pallas import tpu as pltpu
from jax.experimental.pallas import tpu_sc as plsc

B = 4
S = 2048
H = S // 2
NC = 2
NS = 16
NW = NC * NS
GPB = 8
RPG = 8
ACCW = S + 16
BUFW = S + 16 * RPG * 2
NBODY = B * GPB

_k = np.arange(S, dtype=np.float64)[:, None]
_m = np.arange(1, H + 1, dtype=np.float64)[None, :]
_ang = (2.0 * np.pi / S) * _k * _m
_DFT_COS = np.cos(_ang).astype(np.float32)
_DFT_SIN = np.sin(_ang).astype(np.float32)
_INV_COUNTS = np.tile((1.0 / (S - np.arange(S))).astype(np.float32), B)[None, :]


def _sc_body(a_ref, out_ref, bufs, accs, sems):
    wid = 2 * lax.axis_index("s") + lax.axis_index("c")
    zeros16 = jnp.zeros((16,), jnp.float32)

    for buf in bufs[0] + bufs[1]:
        def tail_body(t, carry, buf=buf):
            buf[pl.ds(S + t * 16, 16)] = zeros16
            return carry
        lax.fori_loop(0, (BUFW - S) // 16, tail_body, 0)
    for acc in accs:
        def acc_zero(t, carry, acc=acc):
            acc[pl.ds(t * 16, 16)] = zeros16
            return carry
        lax.fori_loop(0, ACCW // 16, acc_zero, 0)

    def group_rows(b, g):
        c0 = 256 * g
        ln = S - c0
        base = wid + 256 * g
        rows = []
        for j in range(RPG):
            off = (b * S + base + 32 * j) * S + c0
            rows.append((off, c0, ln))
        return rows

    def issue(b, g, parity):
        sem = sems[parity]
        for j, (off, c0, ln) in enumerate(group_rows(b, g)):
            pltpu.async_copy(
                a_ref.at[pl.ds(off, ln)], bufs[parity][j].at[pl.ds(c0, ln)], sem
            )

    def drain(b, g, parity):
        sem = sems[parity]
        for j, (off, c0, ln) in enumerate(group_rows(b, g)):
            pltpu.make_async_copy(
                a_ref.at[pl.ds(off, ln)], bufs[parity][j].at[pl.ds(c0, ln)], sem
            ).wait()

    def process(b, g, parity):
        acc = accs[b]
        bset = bufs[parity]
        i0 = wid + 256 * g
        nchunks = (S - i0 + 15) // 16

        @plsc.parallel_loop(0, nchunks, 1, unroll=4)
        def chunk(kk):
            p = kk * 16
            s0 = bset[0][pl.ds(i0 + p, 16)] + bset[1][pl.ds(i0 + 32 + p, 16)]
            s1 = bset[2][pl.ds(i0 + 64 + p, 16)] + bset[3][pl.ds(i0 + 96 + p, 16)]
            s2 = bset[4][pl.ds(i0 + 128 + p, 16)] + bset[5][pl.ds(i0 + 160 + p, 16)]
            s3 = bset[6][pl.ds(i0 + 192 + p, 16)] + bset[7][pl.ds(i0 + 224 + p, 16)]
            acc[pl.ds(p, 16)] = acc[pl.ds(p, 16)] + ((s0 + s1) + (s2 + s3))

    issue(0, 0, 0)
    issue(0, 1, 1)
    for n in range(NBODY):
        b, g = divmod(n, GPB)
        parity = n % 2
        drain(b, g, parity)
        process(b, g, parity)
        if n + 2 < NBODY:
            b2, g2 = divmod(n + 2, GPB)
            issue(b2, g2, parity)

    for b in range(B):
        pltpu.sync_copy(accs[b].at[pl.ds(0, S)], out_ref.at[wid, b])


@functools.partial(
    pl.kernel,
    out_type=jax.ShapeDtypeStruct((NW, B, S), jnp.float32),
    mesh=plsc.VectorSubcoreMesh(
        core_axis_name="c", subcore_axis_name="s", num_cores=NC, num_subcores=NS
    ),
    scratch_types=[
        [[pltpu.VMEM((BUFW,), jnp.float32) for _ in range(RPG)] for _ in range(2)],
        [pltpu.VMEM((ACCW,), jnp.float32) for _ in range(B)],
        [pltpu.SemaphoreType.DMA for _ in range(2)],
    ],
)
def _sc_diag(a_ref, out_ref, bufs, accs, sems):
    _sc_body(a_ref, out_ref, bufs, accs, sems)


def _reduce_body(p_ref, invc_ref, o_ref):
    o_ref[...] = jnp.sum(p_ref[...], axis=0, keepdims=True) * invc_ref[...]


def _finish_body(w_ref, c_ref, s_ref, o_ref):
    waves = w_ref[...]
    re = jnp.dot(waves, c_ref[...], precision=lax.Precision.HIGHEST)
    im = jnp.dot(waves, s_ref[...], precision=lax.Precision.HIGHEST)
    mag = jnp.sqrt(re * re + im * im)
    mx = jnp.max(mag, axis=1, keepdims=True)
    sm = jnp.sum(mag, axis=1, keepdims=True)
    jv = (1.0 - mx / sm) * (1.0 / B)
    o_ref[...] = jnp.sum(jv, axis=0, keepdims=True)


def kernel(attns):
    a1 = attns.reshape(B * S * S)
    partials = _sc_diag(a1)
    waves_flat = pl.pallas_call(
        _reduce_body,
        out_shape=jax.ShapeDtypeStruct((1, B * S), jnp.float32),
    )(partials.reshape(NW, B * S), jnp.asarray(_INV_COUNTS))
    waves = waves_flat.reshape(B, S)
    out = pl.pallas_call(
        _finish_body,
        out_shape=jax.ShapeDtypeStruct((1, 1), jnp.float32),
    )(waves, jnp.asarray(_DFT_COS), jnp.asarray(_DFT_SIN))
    return out[0, 0]


# speedup vs baseline: 109.7637x; 1.8091x over previous
import functools

import numpy as np
import jax
import jax.numpy as jnp
from jax import lax
from jax.experimental import pallas as pl
from jax.experimental.pallas import tpu as pltpu
from jax.experimental.pallas import tpu_sc as plsc

B = 4
S = 2048
H = S // 2
NC = 2
NS = 16
NW = NC * NS
GPB = 8
RPG = 8
ACCW = S + 16
BUFW = S + 16 * RPG * 2
NBODY = B * GPB

_k = np.arange(S, dtype=np.float64)[:, None]
_m = np.arange(1, H + 1, dtype=np.float64)[None, :]
_ang = (2.0 * np.pi / S) * _k * _m
_DFT_COS = np.cos(_ang).astype(np.float32)
_DFT_SIN = np.sin(_ang).astype(np.float32)
_INV_COUNTS = np.tile((1.0 / (S - np.arange(S))).astype(np.float32), B)[None, :]


def _sc_body(a_ref, out_ref, bufs, accs, sems):
    wid = 2 * lax.axis_index("s") + lax.axis_index("c")
    zeros16 = jnp.zeros((16,), jnp.float32)

    for buf in bufs[0] + bufs[1]:
        def tail_body(t, carry, buf=buf):
            buf[pl.ds(S + t * 16, 16)] = zeros16
            return carry
        lax.fori_loop(0, (BUFW - S) // 16, tail_body, 0)
    for acc in accs:
        def acc_zero(t, carry, acc=acc):
            acc[pl.ds(t * 16, 16)] = zeros16
            return carry
        lax.fori_loop(0, ACCW // 16, acc_zero, 0)

    def group_rows(b, g):
        c0 = 256 * g
        ln = S - c0
        base = wid + 256 * g
        rows = []
        for j in range(RPG):
            rows.append((b * S + base + 32 * j, c0, ln))
        return rows

    def issue(b, g, parity):
        sem = sems[parity]
        for j, (row, c0, ln) in enumerate(group_rows(b, g)):
            pltpu.async_copy(
                a_ref.at[row, pl.ds(c0, ln)], bufs[parity][j].at[pl.ds(c0, ln)], sem
            )

    def drain(b, g, parity):
        sem = sems[parity]
        for j, (row, c0, ln) in enumerate(group_rows(b, g)):
            pltpu.make_async_copy(
                a_ref.at[row, pl.ds(c0, ln)], bufs[parity][j].at[pl.ds(c0, ln)], sem
            ).wait()

    def process(b, g, parity):
        acc = accs[b]
        bset = bufs[parity]
        i0 = wid + 256 * g
        nchunks = (S - i0 + 15) // 16

        @plsc.parallel_loop(0, nchunks, 1, unroll=4)
        def chunk(kk):
            p = kk * 16
            s0 = bset[0][pl.ds(i0 + p, 16)] + bset[1][pl.ds(i0 + 32 + p, 16)]
            s1 = bset[2][pl.ds(i0 + 64 + p, 16)] + bset[3][pl.ds(i0 + 96 + p, 16)]
            s2 = bset[4][pl.ds(i0 + 128 + p, 16)] + bset[5][pl.ds(i0 + 160 + p, 16)]
            s3 = bset[6][pl.ds(i0 + 192 + p, 16)] + bset[7][pl.ds(i0 + 224 + p, 16)]
            acc[pl.ds(p, 16)] = acc[pl.ds(p, 16)] + ((s0 + s1) + (s2 + s3))

    issue(0, 0, 0)
    issue(0, 1, 1)
    for n in range(NBODY):
        b, g = divmod(n, GPB)
        parity = n % 2
        drain(b, g, parity)
        process(b, g, parity)
        if n + 2 < NBODY:
            b2, g2 = divmod(n + 2, GPB)
            issue(b2, g2, parity)

    for b in range(B):
        pltpu.sync_copy(accs[b].at[pl.ds(0, S)], out_ref.at[wid, b])


@functools.partial(
    pl.kernel,
    out_type=jax.ShapeDtypeStruct((NW, B, S), jnp.float32),
    mesh=plsc.VectorSubcoreMesh(
        core_axis_name="c", subcore_axis_name="s", num_cores=NC, num_subcores=NS
    ),
    scratch_types=[
        [[pltpu.VMEM((BUFW,), jnp.float32) for _ in range(RPG)] for _ in range(2)],
        [pltpu.VMEM((ACCW,), jnp.float32) for _ in range(B)],
        [pltpu.SemaphoreType.DMA for _ in range(2)],
    ],
)
def _sc_diag(a_ref, out_ref, bufs, accs, sems):
    _sc_body(a_ref, out_ref, bufs, accs, sems)


def _finish_body(p_ref, invc_ref, c_ref, s_ref, o_ref):
    sums = jnp.sum(p_ref[...], axis=0)
    waves = sums * invc_ref[...]
    re = jnp.dot(waves, c_ref[...], precision=lax.Precision.HIGHEST)
    im = jnp.dot(waves, s_ref[...], precision=lax.Precision.HIGHEST)
    mag = jnp.sqrt(re * re + im * im)
    mx = jnp.max(mag, axis=1, keepdims=True)
    sm = jnp.sum(mag, axis=1, keepdims=True)
    jv = (1.0 - mx / sm) * (1.0 / B)
    o_ref[...] = jnp.sum(jv, axis=0, keepdims=True)


def kernel(attns):
    a2 = attns.reshape(B * S, S)
    partials = _sc_diag(a2)
    out = pl.pallas_call(
        _finish_body,
        out_shape=jax.ShapeDtypeStruct((1, 1), jnp.float32),
    )(
        partials,
        jnp.asarray(_INV_COUNTS[0, :S])[None, :],
        jnp.asarray(_DFT_COS),
        jnp.asarray(_DFT_SIN),
    )
    return out[0, 0]


# speedup vs baseline: 112.1584x; 1.0218x over previous
import functools

import numpy as np
import jax
import jax.numpy as jnp
from jax import lax
from jax.experimental import pallas as pl
from jax.experimental.pallas import tpu as pltpu
from jax.experimental.pallas import tpu_sc as plsc

B = 4
S = 2048
H = S // 2
NC = 2
NS = 16
NW = NC * NS
GPB = 8
RPG = 8
ACCW = S + 32
BUFW = S + 16 * RPG * 2
NBODY = B * GPB
NBUF = 3

_k = np.arange(S, dtype=np.float64)[:, None]
_m = np.arange(1, H + 1, dtype=np.float64)[None, :]
_ang = (2.0 * np.pi / S) * _k * _m
_DFT_COS = np.cos(_ang).astype(np.float32)
_DFT_SIN = np.sin(_ang).astype(np.float32)
_INV_COUNTS = np.tile((1.0 / (S - np.arange(S))).astype(np.float32), B)[None, :]


def _sc_body(a_ref, out_ref, bufs, accs, sems):
    wid = 2 * lax.axis_index("s") + lax.axis_index("c")
    zeros16 = jnp.zeros((16,), jnp.float32)

    for buf in [b for bs in bufs for b in bs]:
        def tail_body(t, carry, buf=buf):
            buf[pl.ds(S + t * 16, 16)] = zeros16
            return carry
        lax.fori_loop(0, (BUFW - S) // 16, tail_body, 0)
    for acc in accs:
        def acc_zero(t, carry, acc=acc):
            acc[pl.ds(t * 16, 16)] = zeros16
            return carry
        lax.fori_loop(0, ACCW // 16, acc_zero, 0)

    def group_slices(b, g, parity, j):
        c0 = 256 * g + 128 * (32 * j // 128)
        ln = S - c0
        row = b * S + wid + 256 * g + 32 * j
        src = a_ref.at[row, pl.ds(c0, ln)]
        dst = bufs[parity][j].at[pl.ds(c0, ln)]
        return src, dst

    def issue(b, g, parity):
        for j in range(RPG):
            src, dst = group_slices(b, g, parity, j)
            pltpu.async_copy(src, dst, sems[parity])

    def drain(b, g, parity):
        for j in range(RPG):
            src, dst = group_slices(b, g, parity, j)
            pltpu.make_async_copy(src, dst, sems[parity]).wait()

    def process(b, g, parity):
        acc = accs[b]
        bset = bufs[parity]
        i0 = wid + 256 * g
        nchunks = (S - i0 + 15) // 16

        @plsc.parallel_loop(0, nchunks, 1, unroll=4)
        def chunk(kk):
            p = kk * 16
            s0 = bset[0][pl.ds(i0 + p, 16)] + bset[1][pl.ds(i0 + 32 + p, 16)]
            s1 = bset[2][pl.ds(i0 + 64 + p, 16)] + bset[3][pl.ds(i0 + 96 + p, 16)]
            s2 = bset[4][pl.ds(i0 + 128 + p, 16)] + bset[5][pl.ds(i0 + 160 + p, 16)]
            s3 = bset[6][pl.ds(i0 + 192 + p, 16)] + bset[7][pl.ds(i0 + 224 + p, 16)]
            acc[pl.ds(p, 16)] = acc[pl.ds(p, 16)] + ((s0 + s1) + (s2 + s3))

    for n0 in range(NBUF):
        b0, g0 = divmod(n0, GPB)
        issue(b0, g0, n0)
    for n in range(NBODY):
        b, g = divmod(n, GPB)
        parity = n % NBUF
        drain(b, g, parity)
        process(b, g, parity)
        if n + NBUF < NBODY:
            b2, g2 = divmod(n + NBUF, GPB)
            issue(b2, g2, parity)

    for b in range(B):
        pltpu.sync_copy(accs[b].at[pl.ds(0, S)], out_ref.at[wid, b])


@functools.partial(
    pl.kernel,
    out_type=jax.ShapeDtypeStruct((NW, B, S), jnp.float32),
    mesh=plsc.VectorSubcoreMesh(
        core_axis_name="c", subcore_axis_name="s", num_cores=NC, num_subcores=NS
    ),
    scratch_types=[
        [[pltpu.VMEM((BUFW,), jnp.float32) for _ in range(RPG)] for _ in range(NBUF)],
        [pltpu.VMEM((ACCW,), jnp.float32) for _ in range(B)],
        [pltpu.SemaphoreType.DMA for _ in range(NBUF)],
    ],
)
def _sc_diag(a_ref, out_ref, bufs, accs, sems):
    _sc_body(a_ref, out_ref, bufs, accs, sems)


def _finish_body(p_ref, invc_ref, c_ref, s_ref, o_ref):
    sums = jnp.sum(p_ref[...], axis=0)
    waves = sums * invc_ref[...]
    re = jnp.dot(waves, c_ref[...], precision=lax.Precision.HIGHEST)
    im = jnp.dot(waves, s_ref[...], precision=lax.Precision.HIGHEST)
    mag = jnp.sqrt(re * re + im * im)
    mx = jnp.max(mag, axis=1, keepdims=True)
    sm = jnp.sum(mag, axis=1, keepdims=True)
    jv = (1.0 - mx / sm) * (1.0 / B)
    o_ref[...] = jnp.sum(jv, axis=0, keepdims=True)


def kernel(attns):
    a2 = attns.reshape(B * S, S)
    partials = _sc_diag(a2)
    out = pl.pallas_call(
        _finish_body,
        out_shape=jax.ShapeDtypeStruct((1, 1), jnp.float32),
    )(
        partials,
        jnp.asarray(_INV_COUNTS[0, :S])[None, :],
        jnp.asarray(_DFT_COS),
        jnp.asarray(_DFT_SIN),
    )
    return out[0, 0]


# speedup vs baseline: 126.3224x; 1.1263x over previous
import functools

import numpy as np
import jax
import jax.numpy as jnp
from jax import lax
from jax.experimental import pallas as pl
from jax.experimental.pallas import tpu as pltpu
from jax.experimental.pallas import tpu_sc as plsc

B = 4
S = 2048
H = S // 2
NC = 2
NS = 16
NW = NC * NS
GPB = 8
RPG = 8
ACCW = S + 32
BUFW = S + 16 * RPG * 2
NBODY = B * GPB
NBUF = 3

_k = np.arange(S, dtype=np.float64)[:, None]
_m = np.arange(1, H + 1, dtype=np.float64)[None, :]
_ang = (2.0 * np.pi / S) * _k * _m
_DFT_COS = np.cos(_ang)
_DFT_SIN = np.sin(_ang)
_INV_COUNTS = np.tile((1.0 / (S - np.arange(S))).astype(np.float32), B)[None, :]


def _sc_body(a_ref, out_ref, bufs, accs, sems):
    wid = 2 * lax.axis_index("s") + lax.axis_index("c")
    zeros16 = jnp.zeros((16,), jnp.float32)

    for buf in [b for bs in bufs for b in bs]:
        def tail_body(t, carry, buf=buf):
            buf[pl.ds(S + t * 16, 16)] = zeros16
            return carry
        lax.fori_loop(0, (BUFW - S) // 16, tail_body, 0)
    for acc in accs:
        def acc_zero(t, carry, acc=acc):
            acc[pl.ds(t * 16, 16)] = zeros16
            return carry
        lax.fori_loop(0, ACCW // 16, acc_zero, 0)

    def group_slices(b, g, parity, j):
        c0 = 256 * g + 128 * (32 * j // 128)
        ln = S - c0
        row = b * S + wid + 256 * g + 32 * j
        src = a_ref.at[row, pl.ds(c0, ln)]
        dst = bufs[parity][j].at[pl.ds(c0, ln)]
        return src, dst

    def issue(b, g, parity):
        for j in range(RPG):
            src, dst = group_slices(b, g, parity, j)
            pltpu.async_copy(src, dst, sems[parity])

    def drain(b, g, parity):
        for j in range(RPG):
            src, dst = group_slices(b, g, parity, j)
            pltpu.make_async_copy(src, dst, sems[parity]).wait()

    def process(b, g, parity):
        acc = accs[b]
        bset = bufs[parity]
        i0 = wid + 256 * g
        nchunks = (S - i0 + 15) // 16

        @plsc.parallel_loop(0, nchunks, 1, unroll=4)
        def chunk(kk):
            p = kk * 16
            s0 = bset[0][pl.ds(i0 + p, 16)] + bset[1][pl.ds(i0 + 32 + p, 16)]
            s1 = bset[2][pl.ds(i0 + 64 + p, 16)] + bset[3][pl.ds(i0 + 96 + p, 16)]
            s2 = bset[4][pl.ds(i0 + 128 + p, 16)] + bset[5][pl.ds(i0 + 160 + p, 16)]
            s3 = bset[6][pl.ds(i0 + 192 + p, 16)] + bset[7][pl.ds(i0 + 224 + p, 16)]
            acc[pl.ds(p, 16)] = acc[pl.ds(p, 16)] + ((s0 + s1) + (s2 + s3))

    for n0 in range(NBUF):
        b0, g0 = divmod(n0, GPB)
        issue(b0, g0, n0)
    for n in range(NBODY):
        b, g = divmod(n, GPB)
        parity = n % NBUF
        drain(b, g, parity)
        process(b, g, parity)
        if n + NBUF < NBODY:
            b2, g2 = divmod(n + NBUF, GPB)
            issue(b2, g2, parity)

    for b in range(B):
        pltpu.sync_copy(accs[b].at[pl.ds(0, S)], out_ref.at[wid, b])


@functools.partial(
    pl.kernel,
    out_type=jax.ShapeDtypeStruct((NW, B, S), jnp.float32),
    mesh=plsc.VectorSubcoreMesh(
        core_axis_name="c", subcore_axis_name="s", num_cores=NC, num_subcores=NS
    ),
    scratch_types=[
        [[pltpu.VMEM((BUFW,), jnp.float32) for _ in range(RPG)] for _ in range(NBUF)],
        [pltpu.VMEM((ACCW,), jnp.float32) for _ in range(B)],
        [pltpu.SemaphoreType.DMA for _ in range(NBUF)],
    ],
)
def _sc_diag(a_ref, out_ref, bufs, accs, sems):
    _sc_body(a_ref, out_ref, bufs, accs, sems)


def _finish_body(p_ref, invc_ref, c_ref, s_ref, o_ref):
    sums = jnp.sum(p_ref[...], axis=0)
    waves = (sums * invc_ref[...]).astype(jnp.bfloat16)
    re = jnp.dot(waves, c_ref[...], preferred_element_type=jnp.float32)
    im = jnp.dot(waves, s_ref[...], preferred_element_type=jnp.float32)
    mag = jnp.sqrt(re * re + im * im)
    mx = jnp.max(mag, axis=1, keepdims=True)
    sm = jnp.sum(mag, axis=1, keepdims=True)
    jv = (1.0 - mx / sm) * (1.0 / B)
    o_ref[...] = jnp.sum(jv, axis=0, keepdims=True)


def kernel(attns):
    a2 = attns.reshape(B * S, S)
    partials = _sc_diag(a2)
    out = pl.pallas_call(
        _finish_body,
        out_shape=jax.ShapeDtypeStruct((1, 1), jnp.float32),
    )(
        partials,
        jnp.asarray(_INV_COUNTS[0, :S])[None, :],
        jnp.asarray(_DFT_COS, dtype=jnp.bfloat16),
        jnp.asarray(_DFT_SIN, dtype=jnp.bfloat16),
    )
    return out[0, 0]


# speedup vs baseline: 128.6957x; 1.0188x over previous
import functools

import numpy as np
import jax
import jax.numpy as jnp
from jax import lax
from jax.experimental import pallas as pl
from jax.experimental.pallas import tpu as pltpu
from jax.experimental.pallas import tpu_sc as plsc

B = 4
S = 2048
H = S // 2
NC = 2
NS = 16
NW = NC * NS
GPB = 8
RPG = 8
ACCW = S + 32
BUFW = S + 16 * RPG * 2
NBODY = B * GPB
NBUF = 3

_k = np.arange(S, dtype=np.float64)[:, None]
_m = np.arange(1, H + 1, dtype=np.float64)[None, :]
_ang = (2.0 * np.pi / S) * _k * _m
_DFT_COS = np.cos(_ang)
_DFT_SIN = np.sin(_ang)
_INV_COUNTS = np.tile((1.0 / (S - np.arange(S))).astype(np.float32), B)[None, :]


def _sc_body(a_ref, out_ref, bufs, accs, drain_dummy, sems):
    wid = 2 * lax.axis_index("s") + lax.axis_index("c")
    zeros16 = jnp.zeros((16,), jnp.float32)

    for buf in [b for bs in bufs for b in bs]:
        def tail_body(t, carry, buf=buf):
            buf[pl.ds(S + t * 16, 16)] = zeros16
            return carry
        lax.fori_loop(0, (BUFW - S) // 16, tail_body, 0)
    for acc in accs:
        def acc_zero(t, carry, acc=acc):
            acc[pl.ds(t * 16, 16)] = zeros16
            return carry
        lax.fori_loop(0, ACCW // 16, acc_zero, 0)

    def group_slices(b, g, parity, j):
        c0 = 256 * g
        ln = S - c0
        row = b * S + wid + 256 * g + 32 * j
        src = a_ref.at[row, pl.ds(c0, ln)]
        dst = bufs[parity][j].at[pl.ds(c0, ln)]
        return src, dst

    def issue(b, g, parity):
        for j in range(RPG):
            src, dst = group_slices(b, g, parity, j)
            pltpu.async_copy(src, dst, sems[parity])

    def drain(b, g, parity):
        ln = S - 256 * g
        pltpu.make_async_copy(
            a_ref.at[pl.ds(0, RPG), pl.ds(0, ln)],
            drain_dummy.at[pl.ds(0, RPG), pl.ds(0, ln)],
            sems[parity],
        ).wait()

    def process(b, g, parity):
        acc = accs[b]
        bset = bufs[parity]
        i0 = wid + 256 * g
        nchunks = (S - i0 + 15) // 16

        @plsc.parallel_loop(0, nchunks, 1, unroll=4)
        def chunk(kk):
            p = kk * 16
            s0 = bset[0][pl.ds(i0 + p, 16)] + bset[1][pl.ds(i0 + 32 + p, 16)]
            s1 = bset[2][pl.ds(i0 + 64 + p, 16)] + bset[3][pl.ds(i0 + 96 + p, 16)]
            s2 = bset[4][pl.ds(i0 + 128 + p, 16)] + bset[5][pl.ds(i0 + 160 + p, 16)]
            s3 = bset[6][pl.ds(i0 + 192 + p, 16)] + bset[7][pl.ds(i0 + 224 + p, 16)]
            acc[pl.ds(p, 16)] = acc[pl.ds(p, 16)] + ((s0 + s1) + (s2 + s3))

    for n0 in range(NBUF):
        b0, g0 = divmod(n0, GPB)
        issue(b0, g0, n0)
    for n in range(NBODY):
        b, g = divmod(n, GPB)
        parity = n % NBUF
        drain(b, g, parity)
        process(b, g, parity)
        if n + NBUF < NBODY:
            b2, g2 = divmod(n + NBUF, GPB)
            issue(b2, g2, parity)

    for b in range(B):
        pltpu.sync_copy(accs[b].at[pl.ds(0, S)], out_ref.at[wid, b])


@functools.partial(
    pl.kernel,
    out_type=jax.ShapeDtypeStruct((NW, B, S), jnp.float32),
    mesh=plsc.VectorSubcoreMesh(
        core_axis_name="c", subcore_axis_name="s", num_cores=NC, num_subcores=NS
    ),
    scratch_types=[
        [[pltpu.VMEM((BUFW,), jnp.float32) for _ in range(RPG)] for _ in range(NBUF)],
        [pltpu.VMEM((ACCW,), jnp.float32) for _ in range(B)],
        pltpu.VMEM((RPG, S), jnp.float32),
        [pltpu.SemaphoreType.DMA for _ in range(NBUF)],
    ],
)
def _sc_diag(a_ref, out_ref, bufs, accs, drain_dummy, sems):
    _sc_body(a_ref, out_ref, bufs, accs, drain_dummy, sems)


def _finish_body(p_ref, invc_ref, c_ref, s_ref, o_ref):
    sums = jnp.sum(p_ref[...], axis=0)
    waves = (sums * invc_ref[...]).astype(jnp.bfloat16)
    re = jnp.dot(waves, c_ref[...], preferred_element_type=jnp.float32)
    im = jnp.dot(waves, s_ref[...], preferred_element_type=jnp.float32)
    mag = jnp.sqrt(re * re + im * im)
    mx = jnp.max(mag, axis=1, keepdims=True)
    sm = jnp.sum(mag, axis=1, keepdims=True)
    jv = (1.0 - mx / sm) * (1.0 / B)
    o_ref[...] = jnp.sum(jv, axis=0, keepdims=True)


def kernel(attns):
    a2 = attns.reshape(B * S, S)
    partials = _sc_diag(a2)
    out = pl.pallas_call(
        _finish_body,
        out_shape=jax.ShapeDtypeStruct((1, 1), jnp.float32),
    )(
        partials,
        jnp.asarray(_INV_COUNTS[0, :S])[None, :],
        jnp.asarray(_DFT_COS, dtype=jnp.bfloat16),
        jnp.asarray(_DFT_SIN, dtype=jnp.bfloat16),
    )
    return out[0, 0]


# speedup vs baseline: 128.7010x; 1.0000x over previous
import functools

import numpy as np
import jax
import jax.numpy as jnp
from jax import lax
from jax.experimental import pallas as pl
from jax.experimental.pallas import tpu as pltpu
from jax.experimental.pallas import tpu_sc as plsc

B = 4
S = 2048
H = S // 2
NC = 2
NS = 16
NW = NC * NS
GPB = 8
RPG = 8
ACCW = S + 32
BUFW = S + 16 * RPG * 2
NBODY = B * GPB
NBUF = 3

_k = np.arange(S, dtype=np.float64)[:, None]
_m = np.arange(1, H + 1, dtype=np.float64)[None, :]
_ang = (2.0 * np.pi / S) * _k * _m
_DFT_COS = np.cos(_ang)
_DFT_SIN = np.sin(_ang)
_INV_COUNTS = (1.0 / (S - np.arange(S))).astype(np.float32)[None, :]


def _sc_body(a_ref, out_ref, bufs, accs, drain_dummy, sems):
    wid = 2 * lax.axis_index("s") + lax.axis_index("c")
    zeros16 = jnp.zeros((16,), jnp.float32)

    for buf in [b for bs in bufs for b in bs]:
        def tail_body(t, carry, buf=buf):
            buf[pl.ds(S + t * 16, 16)] = zeros16
            return carry
        lax.fori_loop(0, (BUFW - S) // 16, tail_body, 0)
    for acc in accs:
        def acc_zero(t, carry, acc=acc):
            acc[pl.ds(t * 16, 16)] = zeros16
            return carry
        lax.fori_loop(0, ACCW // 16, acc_zero, 0)

    def group_slices(b, g, parity, j):
        c0 = 256 * g
        ln = S - c0
        row = b * S + wid + 256 * g + 32 * j
        src = a_ref.at[row, pl.ds(c0, ln)]
        dst = bufs[parity][j].at[pl.ds(c0, ln)]
        return src, dst

    def issue(b, g, parity):
        for j in range(RPG):
            src, dst = group_slices(b, g, parity, j)
            pltpu.async_copy(src, dst, sems[parity])

    def drain(b, g, parity):
        ln = S - 256 * g
        pltpu.make_async_copy(
            a_ref.at[pl.ds(0, RPG), pl.ds(0, ln)],
            drain_dummy.at[pl.ds(0, RPG), pl.ds(0, ln)],
            sems[parity],
        ).wait()

    def process(b, g, parity):
        acc = accs[b]
        bset = bufs[parity]
        i0 = wid + 256 * g
        nchunks = (S - i0 + 15) // 16

        @plsc.parallel_loop(0, nchunks, 1, unroll=4)
        def chunk(kk):
            p = kk * 16
            s0 = bset[0][pl.ds(i0 + p, 16)] + bset[1][pl.ds(i0 + 32 + p, 16)]
            s1 = bset[2][pl.ds(i0 + 64 + p, 16)] + bset[3][pl.ds(i0 + 96 + p, 16)]
            s2 = bset[4][pl.ds(i0 + 128 + p, 16)] + bset[5][pl.ds(i0 + 160 + p, 16)]
            s3 = bset[6][pl.ds(i0 + 192 + p, 16)] + bset[7][pl.ds(i0 + 224 + p, 16)]
            acc[pl.ds(p, 16)] = acc[pl.ds(p, 16)] + ((s0 + s1) + (s2 + s3))

    for n0 in range(NBUF):
        b0, g0 = divmod(n0, GPB)
        issue(b0, g0, n0)
    for n in range(NBODY):
        b, g = divmod(n, GPB)
        parity = n % NBUF
        drain(b, g, parity)
        process(b, g, parity)
        if n + NBUF < NBODY:
            b2, g2 = divmod(n + NBUF, GPB)
            issue(b2, g2, parity)

    for b in range(B):
        pltpu.sync_copy(accs[b].at[pl.ds(0, S)], out_ref.at[wid, b])


@functools.partial(
    pl.kernel,
    out_type=jax.ShapeDtypeStruct((NW, B, S), jnp.float32),
    mesh=plsc.VectorSubcoreMesh(
        core_axis_name="c", subcore_axis_name="s", num_cores=NC, num_subcores=NS
    ),
    scratch_types=[
        [[pltpu.VMEM((BUFW,), jnp.float32) for _ in range(RPG)] for _ in range(NBUF)],
        [pltpu.VMEM((ACCW,), jnp.float32) for _ in range(B)],
        pltpu.VMEM((RPG, S), jnp.float32),
        [pltpu.SemaphoreType.DMA for _ in range(NBUF)],
    ],
)
def _sc_diag(a_ref, out_ref, bufs, accs, drain_dummy, sems):
    _sc_body(a_ref, out_ref, bufs, accs, drain_dummy, sems)


def _finish_body(p_ref, invc_ref, c_ref, s_ref, o_ref):
    sums = jnp.sum(p_ref[...], axis=0)
    waves = (sums * invc_ref[...]).astype(jnp.bfloat16)
    re = jnp.dot(waves, c_ref[...], preferred_element_type=jnp.float32)
    im = jnp.dot(waves, s_ref[...], preferred_element_type=jnp.float32)
    mag = jnp.sqrt(re * re + im * im)
    mx = jnp.max(mag, axis=1, keepdims=True)
    sm = jnp.sum(mag, axis=1, keepdims=True)
    jv = (1.0 - mx / sm) * (1.0 / B)
    o_ref[...] = jnp.sum(jv, axis=0, keepdims=True)


def kernel(attns):
    a2 = attns.reshape(B * S, S)
    partials = _sc_diag(a2)
    out = pl.pallas_call(
        _finish_body,
        out_shape=jax.ShapeDtypeStruct((1, 1), jnp.float32),
    )(
        partials,
        jnp.asarray(_INV_COUNTS),
        jnp.asarray(_DFT_COS, dtype=jnp.bfloat16),
        jnp.asarray(_DFT_SIN, dtype=jnp.bfloat16),
    )
    return out[0, 0]


# speedup vs baseline: 129.3191x; 1.0048x over previous
import functools

import numpy as np
import jax
import jax.numpy as jnp
from jax import lax
from jax.experimental import pallas as pl
from jax.experimental.pallas import tpu as pltpu
from jax.experimental.pallas import tpu_sc as plsc

B = 4
S = 2048
H = S // 2
NC = 2
NS = 16
NW = NC * NS
GPB = 8
RPG = 8
ACCW = S + 32
BUFW = S + 16 * RPG * 2
NBODY = B * GPB
NBUF = 3

_k = np.arange(S, dtype=np.float64)[:, None]
_m = np.arange(1, H + 1, dtype=np.float64)[None, :]
_ang = (2.0 * np.pi / S) * _k * _m
_DFT_COS = np.cos(_ang)
_DFT_SIN = np.sin(_ang)
_INV_COUNTS = (1.0 / (S - np.arange(S))).astype(np.float32)[None, :]


def _sc_body(a_ref, out_ref, bufs, accs, drain_dummy, sems):
    wid = 2 * lax.axis_index("s") + lax.axis_index("c")
    zeros16 = jnp.zeros((16,), jnp.float32)

    for buf in [b for bs in bufs for b in bs]:
        def tail_body(t, carry, buf=buf):
            buf[pl.ds(S + t * 16, 16)] = zeros16
            return carry
        lax.fori_loop(0, (BUFW - S) // 16, tail_body, 0)
    for acc in accs:
        def acc_zero(t, carry, acc=acc):
            acc[pl.ds(t * 16, 16)] = zeros16
            return carry
        lax.fori_loop(0, ACCW // 16, acc_zero, 0)

    def group_slices(b, g, parity, j):
        c0 = 256 * g + 128 * (32 * j // 128)
        ln = S - c0
        row = b * S + wid + 256 * g + 32 * j
        src = a_ref.at[row, pl.ds(c0, ln)]
        dst = bufs[parity][j].at[pl.ds(c0, ln)]
        return src, dst

    def issue(b, g, parity):
        for j in range(RPG):
            src, dst = group_slices(b, g, parity, j)
            pltpu.async_copy(src, dst, sems[parity])

    def drain(b, g, parity):
        for ln in (S - 256 * g, S - 256 * g - 128):
            pltpu.make_async_copy(
                a_ref.at[pl.ds(0, RPG // 2), pl.ds(0, ln)],
                drain_dummy.at[pl.ds(0, RPG // 2), pl.ds(0, ln)],
                sems[parity],
            ).wait()

    def process(b, g, parity):
        acc = accs[b]
        bset = bufs[parity]
        i0 = wid + 256 * g
        nchunks = (S - i0 + 15) // 16

        @plsc.parallel_loop(0, nchunks, 1, unroll=4)
        def chunk(kk):
            p = kk * 16
            s0 = bset[0][pl.ds(i0 + p, 16)] + bset[1][pl.ds(i0 + 32 + p, 16)]
            s1 = bset[2][pl.ds(i0 + 64 + p, 16)] + bset[3][pl.ds(i0 + 96 + p, 16)]
            s2 = bset[4][pl.ds(i0 + 128 + p, 16)] + bset[5][pl.ds(i0 + 160 + p, 16)]
            s3 = bset[6][pl.ds(i0 + 192 + p, 16)] + bset[7][pl.ds(i0 + 224 + p, 16)]
            acc[pl.ds(p, 16)] = acc[pl.ds(p, 16)] + ((s0 + s1) + (s2 + s3))

    for n0 in range(NBUF):
        b0, g0 = divmod(n0, GPB)
        issue(b0, g0, n0)
    for n in range(NBODY):
        b, g = divmod(n, GPB)
        parity = n % NBUF
        drain(b, g, parity)
        process(b, g, parity)
        if n + NBUF < NBODY:
            b2, g2 = divmod(n + NBUF, GPB)
            issue(b2, g2, parity)

    for b in range(B):
        pltpu.sync_copy(accs[b].at[pl.ds(0, S)], out_ref.at[wid, b])


@functools.partial(
    pl.kernel,
    out_type=jax.ShapeDtypeStruct((NW, B, S), jnp.float32),
    mesh=plsc.VectorSubcoreMesh(
        core_axis_name="c", subcore_axis_name="s", num_cores=NC, num_subcores=NS
    ),
    scratch_types=[
        [[pltpu.VMEM((BUFW,), jnp.float32) for _ in range(RPG)] for _ in range(NBUF)],
        [pltpu.VMEM((ACCW,), jnp.float32) for _ in range(B)],
        pltpu.VMEM((RPG, S), jnp.float32),
        [pltpu.SemaphoreType.DMA for _ in range(NBUF)],
    ],
)
def _sc_diag(a_ref, out_ref, bufs, accs, drain_dummy, sems):
    _sc_body(a_ref, out_ref, bufs, accs, drain_dummy, sems)


def _finish_body(p_ref, invc_ref, c_ref, s_ref, o_ref):
    sums = jnp.sum(p_ref[...], axis=0)
    waves = (sums * invc_ref[...]).astype(jnp.bfloat16)
    re = jnp.dot(waves, c_ref[...], preferred_element_type=jnp.float32)
    im = jnp.dot(waves, s_ref[...], preferred_element_type=jnp.float32)
    mag = jnp.sqrt(re * re + im * im)
    mx = jnp.max(mag, axis=1, keepdims=True)
    sm = jnp.sum(mag, axis=1, keepdims=True)
    jv = (1.0 - mx / sm) * (1.0 / B)
    o_ref[...] = jnp.sum(jv, axis=0, keepdims=True)


def kernel(attns):
    a2 = attns.reshape(B * S, S)
    partials = _sc_diag(a2)
    out = pl.pallas_call(
        _finish_body,
        out_shape=jax.ShapeDtypeStruct((1, 1), jnp.float32),
    )(
        partials,
        jnp.asarray(_INV_COUNTS),
        jnp.asarray(_DFT_COS, dtype=jnp.bfloat16),
        jnp.asarray(_DFT_SIN, dtype=jnp.bfloat16),
    )
    return out[0, 0]


# speedup vs baseline: 131.4399x; 1.0164x over previous
import functools

import numpy as np
import jax
import jax.numpy as jnp
from jax import lax
from jax.experimental import pallas as pl
from jax.experimental.pallas import tpu as pltpu
from jax.experimental.pallas import tpu_sc as plsc

B = 4
S = 2048
H = S // 2
NC = 2
NS = 16
NW = NC * NS
GPB = 8
RPG = 8
ACCW = S + 32
BUFW = S + 16 * RPG * 2
NBODY = B * GPB
NBUF = 4

_k = np.arange(S, dtype=np.float64)[:, None]
_m = np.arange(1, H + 1, dtype=np.float64)[None, :]
_ang = (2.0 * np.pi / S) * _k * _m
_DFT_COS = np.cos(_ang)
_DFT_SIN = np.sin(_ang)
_INV_COUNTS = (1.0 / (S - np.arange(S))).astype(np.float32)[None, :]


def _sc_body(a_ref, out_ref, bufs, accs, drain_dummy, sems):
    wid = 2 * lax.axis_index("s") + lax.axis_index("c")
    zeros16 = jnp.zeros((16,), jnp.float32)

    for buf in [b for bs in bufs for b in bs]:
        def tail_body(t, carry, buf=buf):
            buf[pl.ds(S + t * 16, 16)] = zeros16
            return carry
        lax.fori_loop(0, (BUFW - S) // 16, tail_body, 0)
    for acc in accs:
        def acc_zero(t, carry, acc=acc):
            acc[pl.ds(t * 16, 16)] = zeros16
            return carry
        lax.fori_loop(0, ACCW // 16, acc_zero, 0)

    def group_slices(b, g, parity, j):
        c0 = 256 * g + 128 * (32 * j // 128)
        ln = S - c0
        row = b * S + wid + 256 * g + 32 * j
        src = a_ref.at[row, pl.ds(c0, ln)]
        dst = bufs[parity][j].at[pl.ds(c0, ln)]
        return src, dst

    def issue(b, g, parity):
        for j in range(RPG):
            src, dst = group_slices(b, g, parity, j)
            pltpu.async_copy(src, dst, sems[parity])

    def drain(b, g, parity):
        for ln in (S - 256 * g, S - 256 * g - 128):
            pltpu.make_async_copy(
                a_ref.at[pl.ds(0, RPG // 2), pl.ds(0, ln)],
                drain_dummy.at[pl.ds(0, RPG // 2), pl.ds(0, ln)],
                sems[parity],
            ).wait()

    def process(b, g, parity):
        acc = accs[b]
        bset = bufs[parity]
        i0 = wid + 256 * g
        nchunks = (S - i0 + 15) // 16

        @plsc.parallel_loop(0, nchunks, 1, unroll=4)
        def chunk(kk):
            p = kk * 16
            s0 = bset[0][pl.ds(i0 + p, 16)] + bset[1][pl.ds(i0 + 32 + p, 16)]
            s1 = bset[2][pl.ds(i0 + 64 + p, 16)] + bset[3][pl.ds(i0 + 96 + p, 16)]
            s2 = bset[4][pl.ds(i0 + 128 + p, 16)] + bset[5][pl.ds(i0 + 160 + p, 16)]
            s3 = bset[6][pl.ds(i0 + 192 + p, 16)] + bset[7][pl.ds(i0 + 224 + p, 16)]
            acc[pl.ds(p, 16)] = acc[pl.ds(p, 16)] + ((s0 + s1) + (s2 + s3))

    for n0 in range(NBUF):
        b0, g0 = divmod(n0, GPB)
        issue(b0, g0, n0)
    for n in range(NBODY):
        b, g = divmod(n, GPB)
        parity = n % NBUF
        drain(b, g, parity)
        process(b, g, parity)
        if n + NBUF < NBODY:
            b2, g2 = divmod(n + NBUF, GPB)
            issue(b2, g2, parity)

    for b in range(B):
        pltpu.sync_copy(accs[b].at[pl.ds(0, S)], out_ref.at[wid, b])


@functools.partial(
    pl.kernel,
    out_type=jax.ShapeDtypeStruct((NW, B, S), jnp.float32),
    mesh=plsc.VectorSubcoreMesh(
        core_axis_name="c", subcore_axis_name="s", num_cores=NC, num_subcores=NS
    ),
    scratch_types=[
        [[pltpu.VMEM((BUFW,), jnp.float32) for _ in range(RPG)] for _ in range(NBUF)],
        [pltpu.VMEM((ACCW,), jnp.float32) for _ in range(B)],
        pltpu.VMEM((RPG, S), jnp.float32),
        [pltpu.SemaphoreType.DMA for _ in range(NBUF)],
    ],
)
def _sc_diag(a_ref, out_ref, bufs, accs, drain_dummy, sems):
    _sc_body(a_ref, out_ref, bufs, accs, drain_dummy, sems)


def _finish_body(p_ref, invc_ref, c_ref, s_ref, o_ref):
    sums = jnp.sum(p_ref[...], axis=0)
    waves = (sums * invc_ref[...]).astype(jnp.bfloat16)
    re = jnp.dot(waves, c_ref[...], preferred_element_type=jnp.float32)
    im = jnp.dot(waves, s_ref[...], preferred_element_type=jnp.float32)
    mag = jnp.sqrt(re * re + im * im)
    mx = jnp.max(mag, axis=1, keepdims=True)
    sm = jnp.sum(mag, axis=1, keepdims=True)
    jv = (1.0 - mx / sm) * (1.0 / B)
    o_ref[...] = jnp.sum(jv, axis=0, keepdims=True)


def kernel(attns):
    a2 = attns.reshape(B * S, S)
    partials = _sc_diag(a2)
    out = pl.pallas_call(
        _finish_body,
        out_shape=jax.ShapeDtypeStruct((1, 1), jnp.float32),
    )(
        partials,
        jnp.asarray(_INV_COUNTS),
        jnp.asarray(_DFT_COS, dtype=jnp.bfloat16),
        jnp.asarray(_DFT_SIN, dtype=jnp.bfloat16),
    )
    return out[0, 0]
